# baseline (speedup 1.0000x reference)
import functools
import math

import jax
import jax.numpy as jnp
from jax import lax
from jax.experimental import pallas as pl
from jax.experimental.pallas import tpu as pltpu
from jax.experimental.pallas import tpu_sc as plsc

B, T, DIN, DH, DM, K = 16, 4096, 32, 64, 128, 32
SEG = T // K
PACK = 4
TP = T // PACK
SEGP = SEG // PACK
DIN4 = DIN * PACK
DH4 = DH * PACK

TCH = T // 2
NCHUNK = 4
TCK = TCH // NCHUNK
RCK = TCK // PACK


def _repack_body(x_hbm, out_hbm, a_ref, b_ref):
    c = lax.axis_index("c")
    s = lax.axis_index("s")

    def body(r, carry):
        for cc in range(8):
            b_ref[r, pl.ds(cc * 16, 16)] = a_ref[PACK * r + cc // 2,
                                                 pl.ds((cc % 2) * 16, 16)]
        return carry

    for q in range(NCHUNK):
        in_off = pl.multiple_of(c * TCH + q * TCK, TCK)
        out_off = pl.multiple_of(c * (TCH // PACK) + q * RCK, RCK)
        pltpu.sync_copy(x_hbm.at[s, pl.ds(in_off, TCK), :], a_ref)
        lax.fori_loop(0, RCK, body, 0)
        pltpu.sync_copy(b_ref, out_hbm.at[s, pl.ds(out_off, RCK), :])


_repack = functools.partial(
    pl.kernel,
    mesh=plsc.VectorSubcoreMesh(core_axis_name="c", subcore_axis_name="s"),
    out_type=jax.ShapeDtypeStruct((B, TP, DIN4), jnp.float32),
    scratch_types=[
        pltpu.VMEM((TCK, DIN), jnp.float32),
        pltpu.VMEM((RCK, DIN4), jnp.float32),
    ],
)(_repack_body)


def _fused_kernel(x_ref, w1_ref, b1_ref, w2_ref, b2_ref, g_ref, out_ref):
    x = x_ref[0]
    h = jnp.dot(x, w1_ref[...], preferred_element_type=jnp.float32) + b1_ref[...]
    h = 0.5 * h * (1.0 + jax.lax.erf(h * (1.0 / math.sqrt(2.0))))
    pooled = h.reshape(K, SEGP, DH4).sum(axis=1) * (1.0 / SEG)
    out = jnp.dot(pooled, w2_ref[...], preferred_element_type=jnp.float32)
    g = jnp.tanh(g_ref[0, 0])
    out_ref[0] = g * (out + b2_ref[...])


@jax.jit
def kernel(imu_seq, W1, b1, W2, b2, gate):
    x4 = _repack(imu_seq)
    W1b = jnp.kron(jnp.eye(PACK, dtype=W1.dtype), W1)
    b1t = jnp.tile(b1, PACK).reshape(1, DH4)
    W2s = jnp.concatenate([W2] * PACK, axis=0)
    out = pl.pallas_call(
        _fused_kernel,
        grid=(B,),
        in_specs=[
            pl.BlockSpec((1, TP, DIN4), lambda b: (b, 0, 0)),
            pl.BlockSpec((DIN4, DH4), lambda b: (0, 0)),
            pl.BlockSpec((1, DH4), lambda b: (0, 0)),
            pl.BlockSpec((DH4, DM), lambda b: (0, 0)),
            pl.BlockSpec((1, DM), lambda b: (0, 0)),
            pl.BlockSpec((1, 1), lambda b: (0, 0)),
        ],
        out_specs=pl.BlockSpec((1, K, DM), lambda b: (b, 0, 0)),
        out_shape=jax.ShapeDtypeStruct((B, K, DM), jnp.float32),
        compiler_params=pltpu.CompilerParams(
            dimension_semantics=("arbitrary",),
        ),
    )(
        x4,
        W1b,
        b1t,
        W2s,
        b2.reshape(1, DM),
        gate.reshape(1, 1),
    )
    return out


# speedup vs baseline: 1.1336x; 1.1336x over previous
import functools
import math

import jax
import jax.numpy as jnp
from jax import lax
from jax.experimental import pallas as pl
from jax.experimental.pallas import tpu as pltpu
from jax.experimental.pallas import tpu_sc as plsc

B, T, DIN, DH, DM, K = 16, 4096, 32, 64, 128, 32
SEG = T // K
PACK = 4
TP = T // PACK
SEGP = SEG // PACK
DIN4 = DIN * PACK
DH4 = DH * PACK

TCH = T // 2
NCHUNK = 8
TCK = TCH // NCHUNK
RCK = TCK // PACK


def _repack_body(x_hbm, out_hbm, a0, a1, b0, b1, si0, si1, so0, so1):
    c = lax.axis_index("c")
    s = lax.axis_index("s")
    abufs, bbufs = (a0, a1), (b0, b1)
    sins, souts = (si0, si1), (so0, so1)
    base = c * TCH

    def start_in(q):
        off = pl.multiple_of(base + q * TCK, TCK)
        return pltpu.async_copy(x_hbm.at[s, pl.ds(off, TCK), :],
                                abufs[q % 2], sins[q % 2])

    def start_out(q):
        off = pl.multiple_of((base + q * TCK) // PACK, RCK)
        return pltpu.async_copy(bbufs[q % 2],
                                out_hbm.at[s, pl.ds(off, RCK), :],
                                souts[q % 2])

    cin = start_in(0)
    outs = [None] * NCHUNK
    for q in range(NCHUNK):
        nxt = start_in(q + 1) if q + 1 < NCHUNK else None
        cin.wait()
        if q >= 2:
            outs[q - 2].wait()
        a_ref, b_ref = abufs[q % 2], bbufs[q % 2]

        def body(r, carry):
            for rr in range(4):
                row = 4 * r + rr
                for cc in range(8):
                    b_ref[row, pl.ds(cc * 16, 16)] = (
                        a_ref[PACK * row + cc // 2, pl.ds((cc % 2) * 16, 16)])
            return carry

        lax.fori_loop(0, RCK // 4, body, 0)
        outs[q] = start_out(q)
        cin = nxt
    outs[NCHUNK - 2].wait()
    outs[NCHUNK - 1].wait()


_repack = functools.partial(
    pl.kernel,
    mesh=plsc.VectorSubcoreMesh(core_axis_name="c", subcore_axis_name="s"),
    out_type=jax.ShapeDtypeStruct((B, TP, DIN4), jnp.float32),
    scratch_types=[
        pltpu.VMEM((TCK, DIN), jnp.float32),
        pltpu.VMEM((TCK, DIN), jnp.float32),
        pltpu.VMEM((RCK, DIN4), jnp.float32),
        pltpu.VMEM((RCK, DIN4), jnp.float32),
        pltpu.SemaphoreType.DMA,
        pltpu.SemaphoreType.DMA,
        pltpu.SemaphoreType.DMA,
        pltpu.SemaphoreType.DMA,
    ],
)(_repack_body)


def _fused_kernel(x_ref, w1_ref, b1_ref, w2_ref, b2_ref, g_ref, out_ref):
    x = x_ref[0]
    h = jnp.dot(x, w1_ref[...], preferred_element_type=jnp.float32) + b1_ref[...]
    h = 0.5 * h * (1.0 + jax.lax.erf(h * (1.0 / math.sqrt(2.0))))
    pooled = h.reshape(K, SEGP, DH4).sum(axis=1) * (1.0 / SEG)
    out = jnp.dot(pooled, w2_ref[...], preferred_element_type=jnp.float32)
    g = jnp.tanh(g_ref[0, 0])
    out_ref[0] = g * (out + b2_ref[...])


@jax.jit
def kernel(imu_seq, W1, b1, W2, b2, gate):
    x4 = _repack(imu_seq)
    W1b = jnp.kron(jnp.eye(PACK, dtype=W1.dtype), W1)
    b1t = jnp.tile(b1, PACK).reshape(1, DH4)
    W2s = jnp.concatenate([W2] * PACK, axis=0)
    out = pl.pallas_call(
        _fused_kernel,
        grid=(B,),
        in_specs=[
            pl.BlockSpec((1, TP, DIN4), lambda b: (b, 0, 0)),
            pl.BlockSpec((DIN4, DH4), lambda b: (0, 0)),
            pl.BlockSpec((1, DH4), lambda b: (0, 0)),
            pl.BlockSpec((DH4, DM), lambda b: (0, 0)),
            pl.BlockSpec((1, DM), lambda b: (0, 0)),
            pl.BlockSpec((1, 1), lambda b: (0, 0)),
        ],
        out_specs=pl.BlockSpec((1, K, DM), lambda b: (b, 0, 0)),
        out_shape=jax.ShapeDtypeStruct((B, K, DM), jnp.float32),
        compiler_params=pltpu.CompilerParams(
            dimension_semantics=("arbitrary",),
        ),
    )(
        x4,
        W1b,
        b1t,
        W2s,
        b2.reshape(1, DM),
        gate.reshape(1, 1),
    )
    return out


# speedup vs baseline: 2.0466x; 1.8055x over previous
import functools
import math

import jax
import jax.numpy as jnp
from jax.experimental import pallas as pl
from jax.experimental.pallas import tpu as pltpu

B, T, DIN, DH, DM, K = 16, 4096, 32, 64, 128, 32
SEG = T // K
NS = 4
NSTEP = B // NS


def _mlp_pool(x, w1, b1, w2):
    x2 = x.reshape(T, DIN)
    h = jnp.dot(x2, w1, preferred_element_type=jnp.float32) + b1
    h = 0.5 * h * (1.0 + jax.lax.erf(h * (1.0 / math.sqrt(2.0))))
    pooled = h.reshape(K, SEG, DH).sum(axis=1) * (1.0 / SEG)
    return jnp.dot(pooled, w2, preferred_element_type=jnp.float32)


def _fused_kernel(x0_ref, x1_ref, x2_ref, x3_ref,
                  w1_ref, b1_ref, w2_ref, b2_ref, g_ref,
                  o0_ref, o1_ref, o2_ref, o3_ref):
    w1 = w1_ref[...]
    b1 = b1_ref[...]
    w2 = w2_ref[...]
    b2 = b2_ref[...]
    g = jnp.tanh(g_ref[0, 0])
    for x_ref, o_ref in ((x0_ref, o0_ref), (x1_ref, o1_ref),
                         (x2_ref, o2_ref), (x3_ref, o3_ref)):
        out = _mlp_pool(x_ref[...], w1, b1, w2)
        o_ref[...] = (g * (out + b2)).reshape(1, K, DM)


@jax.jit
def kernel(imu_seq, W1, b1, W2, b2, gate):
    outs = pl.pallas_call(
        _fused_kernel,
        grid=(NSTEP,),
        in_specs=[
            pl.BlockSpec((1, T, DIN), lambda j, i=i: (j + NSTEP * i, 0, 0))
            for i in range(NS)
        ] + [
            pl.BlockSpec((DIN, DH), lambda j: (0, 0)),
            pl.BlockSpec((1, DH), lambda j: (0, 0)),
            pl.BlockSpec((DH, DM), lambda j: (0, 0)),
            pl.BlockSpec((1, DM), lambda j: (0, 0)),
            pl.BlockSpec((1, 1), lambda j: (0, 0)),
        ],
        out_specs=[
            pl.BlockSpec((1, K, DM), lambda j: (j, 0, 0)) for i in range(NS)
        ],
        out_shape=[
            jax.ShapeDtypeStruct((NSTEP, K, DM), jnp.float32)
            for i in range(NS)
        ],
        compiler_params=pltpu.CompilerParams(
            dimension_semantics=("arbitrary",),
        ),
    )(
        imu_seq,
        imu_seq,
        imu_seq,
        imu_seq,
        W1,
        b1.reshape(1, DH),
        W2,
        b2.reshape(1, DM),
        gate.reshape(1, 1),
    )
    return jnp.concatenate(outs, axis=0)


# speedup vs baseline: 2.0718x; 1.0123x over previous
import functools
import math

import jax
import jax.numpy as jnp
from jax.experimental import pallas as pl
from jax.experimental.pallas import tpu as pltpu

B, T, DIN, DH, DM, K = 16, 4096, 32, 64, 128, 32
SEG = T // K
GB = 4
NSTEP = B // (2 * GB)


def _mlp_pool(x, w1, b1, w2):
    x2 = x.reshape(GB * T, DIN)
    h = jnp.dot(x2, w1, preferred_element_type=jnp.float32) + b1
    h = 0.5 * h * (1.0 + jax.lax.erf(h * (1.0 / math.sqrt(2.0))))
    pooled = h.reshape(GB * K, SEG, DH).sum(axis=1) * (1.0 / SEG)
    return jnp.dot(pooled, w2, preferred_element_type=jnp.float32)


def _fused_kernel(x0_ref, x1_ref, w1_ref, b1_ref, w2_ref, b2_ref, g_ref,
                  o0_ref, o1_ref):
    w1 = w1_ref[...]
    b1 = b1_ref[...]
    w2 = w2_ref[...]
    g = jnp.tanh(g_ref[0, 0])
    out0 = _mlp_pool(x0_ref[...], w1, b1, w2)
    o0_ref[...] = (g * (out0 + b2_ref[...])).reshape(GB, K, DM)
    out1 = _mlp_pool(x1_ref[...], w1, b1, w2)
    o1_ref[...] = (g * (out1 + b2_ref[...])).reshape(GB, K, DM)


@jax.jit
def kernel(imu_seq, W1, b1, W2, b2, gate):
    outs = pl.pallas_call(
        _fused_kernel,
        grid=(NSTEP,),
        in_specs=[
            pl.BlockSpec((GB, T, DIN), lambda j: (j, 0, 0)),
            pl.BlockSpec((GB, T, DIN), lambda j: (j + NSTEP, 0, 0)),
            pl.BlockSpec((DIN, DH), lambda j: (0, 0)),
            pl.BlockSpec((1, DH), lambda j: (0, 0)),
            pl.BlockSpec((DH, DM), lambda j: (0, 0)),
            pl.BlockSpec((1, DM), lambda j: (0, 0)),
            pl.BlockSpec((1, 1), lambda j: (0, 0)),
        ],
        out_specs=[
            pl.BlockSpec((GB, K, DM), lambda j: (j, 0, 0)),
            pl.BlockSpec((GB, K, DM), lambda j: (j, 0, 0)),
        ],
        out_shape=[
            jax.ShapeDtypeStruct((B // 2, K, DM), jnp.float32),
            jax.ShapeDtypeStruct((B // 2, K, DM), jnp.float32),
        ],
        compiler_params=pltpu.CompilerParams(
            dimension_semantics=("arbitrary",),
        ),
    )(
        imu_seq,
        imu_seq,
        W1,
        b1.reshape(1, DH),
        W2,
        b2.reshape(1, DM),
        gate.reshape(1, 1),
    )
    return jnp.concatenate(outs, axis=0)


# speedup vs baseline: 2.1098x; 1.0184x over previous
import functools
import math

import jax
import jax.numpy as jnp
from jax.experimental import pallas as pl
from jax.experimental.pallas import tpu as pltpu

B, T, DIN, DH, DM, K = 16, 4096, 32, 64, 128, 32
SEG = T // K
GB = 2
NSTEP = B // (2 * GB)


def _mlp_pool(x, w1, b1, w2):
    x2 = x.reshape(GB * T, DIN)
    h = jnp.dot(x2, w1, preferred_element_type=jnp.float32) + b1
    h = 0.5 * h * (1.0 + jax.lax.erf(h * (1.0 / math.sqrt(2.0))))
    pooled = h.reshape(GB * K, SEG, DH).sum(axis=1) * (1.0 / SEG)
    return jnp.dot(pooled, w2, preferred_element_type=jnp.float32)


def _fused_kernel(x0_ref, x1_ref, w1_ref, b1_ref, w2_ref, b2_ref, g_ref,
                  o0_ref, o1_ref):
    w1 = w1_ref[...]
    b1 = b1_ref[...]
    w2 = w2_ref[...]
    g = jnp.tanh(g_ref[0, 0])
    out0 = _mlp_pool(x0_ref[...], w1, b1, w2)
    o0_ref[...] = (g * (out0 + b2_ref[...])).reshape(GB, K, DM)
    out1 = _mlp_pool(x1_ref[...], w1, b1, w2)
    o1_ref[...] = (g * (out1 + b2_ref[...])).reshape(GB, K, DM)


@jax.jit
def kernel(imu_seq, W1, b1, W2, b2, gate):
    outs = pl.pallas_call(
        _fused_kernel,
        grid=(NSTEP,),
        in_specs=[
            pl.BlockSpec((GB, T, DIN), lambda j: (j, 0, 0)),
            pl.BlockSpec((GB, T, DIN), lambda j: (j + NSTEP, 0, 0)),
            pl.BlockSpec((DIN, DH), lambda j: (0, 0)),
            pl.BlockSpec((1, DH), lambda j: (0, 0)),
            pl.BlockSpec((DH, DM), lambda j: (0, 0)),
            pl.BlockSpec((1, DM), lambda j: (0, 0)),
            pl.BlockSpec((1, 1), lambda j: (0, 0)),
        ],
        out_specs=[
            pl.BlockSpec((GB, K, DM), lambda j: (j, 0, 0)),
            pl.BlockSpec((GB, K, DM), lambda j: (j, 0, 0)),
        ],
        out_shape=[
            jax.ShapeDtypeStruct((B // 2, K, DM), jnp.float32),
            jax.ShapeDtypeStruct((B // 2, K, DM), jnp.float32),
        ],
        compiler_params=pltpu.CompilerParams(
            dimension_semantics=("arbitrary",),
        ),
    )(
        imu_seq,
        imu_seq,
        W1,
        b1.reshape(1, DH),
        W2,
        b2.reshape(1, DM),
        gate.reshape(1, 1),
    )
    return jnp.concatenate(outs, axis=0)
